# Initial kernel scaffold; baseline (speedup 1.0000x reference)
import jax
import jax.numpy as jnp
from jax.experimental import pallas as pl


def kernel(x, W_linear, W_cross, bias):
    raise NotImplementedError("write your pallas kernel here")


# speedup vs baseline: 38.6954x; 38.6954x over previous
import functools

import jax
import jax.numpy as jnp
import numpy as np
from jax import lax
from jax.experimental import pallas as pl
from jax.experimental.pallas import tpu as pltpu
from jax.experimental.pallas import tpu_sc as plsc

F = 26
D = 16
B = 4096
TOTAL = 26000
ROW = 512
LIN_OFF = F * D
PAIRS = [(i, j) for i in range(F) for j in range(i + 1, F)]

_SC = plsc.get_sparse_core_info()
NC, NS = _SC.num_cores, _SC.num_subcores
NW = NC * NS
SPT = B // NW
CH = 4
NCHUNK = SPT // CH
IDXC = CH * F


def _lane_sum(v, lanes):
    for step in (8, 4, 2, 1):
        v = v + v.at[lanes ^ step].get(mode="promise_in_bounds", unique_indices=True)
    return v


def _sc_body(p_hbm, idx_hbm, bias_hbm, out_hbm, idx_v, rows_v, part_v, out_v,
             bias_v, sem):
    wid = lax.axis_index("s") * NC + lax.axis_index("c")
    base = wid * SPT
    pltpu.sync_copy(idx_hbm.at[pl.ds(base * F, SPT * F)], idx_v)
    pltpu.sync_copy(bias_hbm, bias_v)

    def chunk(c, carry):
        pltpu.async_copy(p_hbm.at[idx_v.at[pl.ds(c * IDXC, IDXC)]], rows_v, sem).wait()

        def sample(s, carry2):
            r0 = s * F
            acc = [jnp.zeros((D,), jnp.float32) for _ in range(4)]
            t = 0
            for i in range(F):
                acc[t % 4] = acc[t % 4] + rows_v[r0 + i, pl.ds(LIN_OFF, D)]
                t += 1
            for (i, j) in PAIRS:
                a = rows_v[r0 + i, pl.ds(j * D, D)]
                b = rows_v[r0 + j, pl.ds(i * D, D)]
                acc[t % 4] = acc[t % 4] + a * b
                t += 1
            part_v[c * CH + s] = (acc[0] + acc[1]) + (acc[2] + acc[3])
            return carry2

        return lax.fori_loop(0, CH, sample, carry)

    lax.fori_loop(0, NCHUNK, chunk, 0)

    lanes = lax.iota(jnp.int32, 16)
    bias_b = _lane_sum(bias_v[...], lanes)
    for k in range(SPT // D):
        vec = jnp.zeros((D,), jnp.float32)
        for l in range(D):
            s = _lane_sum(part_v[k * D + l], lanes)
            vec = jnp.where(lanes == l, s, vec)
        vec = vec + bias_b
        out_v[pl.ds(k * D, D)] = 1.0 / (1.0 + jnp.exp(-vec))
    pltpu.sync_copy(out_v, out_hbm.at[pl.ds(base, SPT)])


@functools.partial(
    pl.kernel,
    mesh=plsc.VectorSubcoreMesh(core_axis_name="c", subcore_axis_name="s"),
    out_type=jax.ShapeDtypeStruct((B,), jnp.float32),
    scratch_types=[
        pltpu.VMEM((SPT * F,), jnp.int32),
        pltpu.VMEM((IDXC, ROW), jnp.float32),
        pltpu.VMEM((SPT, D), jnp.float32),
        pltpu.VMEM((SPT,), jnp.float32),
        pltpu.VMEM((D,), jnp.float32),
        pltpu.SemaphoreType.DMA,
    ],
)
def _sc_kernel(p_hbm, idx_hbm, bias_hbm, out_hbm, idx_v, rows_v, part_v, out_v,
               bias_v, sem):
    _sc_body(p_hbm, idx_hbm, bias_hbm, out_hbm, idx_v, rows_v, part_v, out_v,
             bias_v, sem)


def kernel(x, W_linear, W_cross, bias):
    offs = jnp.arange(F, dtype=jnp.int32) * 1000
    idx = (x.astype(jnp.int32) + offs[None, :]).reshape(-1)
    p = jnp.transpose(W_cross, (1, 0, 2)).reshape(TOTAL, F * D)
    linpad = jnp.concatenate(
        [W_linear.astype(jnp.float32),
         jnp.zeros((TOTAL, ROW - F * D - 1), jnp.float32)], axis=1)
    p_aug = jnp.concatenate([p, linpad], axis=1)
    bias_pad = jnp.concatenate([bias.astype(jnp.float32), jnp.zeros((D - 1,), jnp.float32)])
    out = _sc_kernel(p_aug, idx, bias_pad)
    return out.reshape(B, 1)


# speedup vs baseline: 46.7710x; 1.2087x over previous
import functools

import jax
import jax.numpy as jnp
import numpy as np
from jax import lax
from jax.experimental import pallas as pl
from jax.experimental.pallas import tpu as pltpu
from jax.experimental.pallas import tpu_sc as plsc

F = 26
D = 16
B = 4096
TOTAL = 26000
ROW = 512
LIN_OFF = F * D
PAIRS = [(i, j) for i in range(F) for j in range(i + 1, F)]

_SC = plsc.get_sparse_core_info()
NC, NS = _SC.num_cores, _SC.num_subcores
NW = NC * NS
SPT = B // NW
CH = 4
NCHUNK = SPT // CH
IDXC = CH * F


def _lane_sum(v, lanes):
    for step in (8, 4, 2, 1):
        v = v + v.at[lanes ^ step].get(mode="promise_in_bounds", unique_indices=True)
    return v


def _sc_body(p_hbm, idx_hbm, bias_hbm, out_hbm, idx_v, rows_v,
             part_v, out_v, bias_v, sem0, sem1):
    wid = lax.axis_index("s") * NC + lax.axis_index("c")
    base = wid * SPT
    pltpu.sync_copy(idx_hbm.at[pl.ds(base * F, SPT * F)], idx_v)
    pltpu.sync_copy(bias_hbm, bias_v)

    def start(c, half, sem):
        pltpu.async_copy(
            p_hbm.at[idx_v.at[pl.ds(c * IDXC, IDXC)]],
            rows_v.at[pl.ds(half * IDXC, IDXC)], sem)

    start(0, 0, sem0)
    lanes = lax.iota(jnp.int32, 16)
    bias_b = _lane_sum(bias_v[...], lanes)

    def chunk(c, carry):
        nxt = c + 1

        @pl.when(jnp.logical_and(nxt < NCHUNK, nxt % 2 == 0))
        def _():
            start(nxt, 0, sem0)

        @pl.when(jnp.logical_and(nxt < NCHUNK, nxt % 2 == 1))
        def _():
            start(nxt, 1, sem1)

        @pl.when(c % 2 == 0)
        def _():
            pltpu.make_async_copy(
                p_hbm, rows_v.at[pl.ds(0, IDXC)], sem0).wait()

        @pl.when(c % 2 == 1)
        def _():
            pltpu.make_async_copy(
                p_hbm, rows_v.at[pl.ds(0, IDXC)], sem1).wait()

        roff = (c % 2) * IDXC

        def sample(s, carry2):
            r0 = roff + s * F
            acc = [jnp.zeros((D,), jnp.float32) for _ in range(2)]
            t = 0
            for i in range(F):
                acc[t % 2] = acc[t % 2] + rows_v[r0 + i, pl.ds(LIN_OFF, D)]
                t += 1
            for (i, j) in PAIRS:
                a = rows_v[r0 + i, pl.ds(j * D, D)]
                b = rows_v[r0 + j, pl.ds(i * D, D)]
                acc[t % 2] = acc[t % 2] + a * b
                t += 1
            part_v[(c % 4) * CH + s] = acc[0] + acc[1]
            return carry2

        lax.fori_loop(0, CH, sample, carry)

        @pl.when(c % 4 == 3)
        def _():
            vec = jnp.zeros((D,), jnp.float32)
            for l in range(D):
                sm = _lane_sum(part_v[l], lanes)
                vec = jnp.where(lanes == l, sm, vec)
            vec = vec + bias_b
            out_v[pl.ds((c // 4) * D, D)] = 1.0 / (1.0 + jnp.exp(-vec))

        return carry

    lax.fori_loop(0, NCHUNK, chunk, 0)
    pltpu.sync_copy(out_v, out_hbm.at[pl.ds(base, SPT)])


@functools.partial(
    pl.kernel,
    mesh=plsc.VectorSubcoreMesh(core_axis_name="c", subcore_axis_name="s"),
    out_type=jax.ShapeDtypeStruct((B,), jnp.float32),
    scratch_types=[
        pltpu.VMEM((SPT * F,), jnp.int32),
        pltpu.VMEM((2 * IDXC, ROW), jnp.float32),
        pltpu.VMEM((4 * CH, D), jnp.float32),
        pltpu.VMEM((SPT,), jnp.float32),
        pltpu.VMEM((D,), jnp.float32),
        pltpu.SemaphoreType.DMA,
        pltpu.SemaphoreType.DMA,
    ],
)
def _sc_kernel(p_hbm, idx_hbm, bias_hbm, out_hbm, idx_v, rows_v,
               part_v, out_v, bias_v, sem0, sem1):
    _sc_body(p_hbm, idx_hbm, bias_hbm, out_hbm, idx_v, rows_v,
             part_v, out_v, bias_v, sem0, sem1)


def kernel(x, W_linear, W_cross, bias):
    offs = jnp.arange(F, dtype=jnp.int32) * 1000
    idx = (x.astype(jnp.int32) + offs[None, :]).reshape(-1)
    p = jnp.transpose(W_cross, (1, 0, 2)).reshape(TOTAL, F * D)
    linpad = jnp.concatenate(
        [W_linear.astype(jnp.float32),
         jnp.zeros((TOTAL, ROW - F * D - 1), jnp.float32)], axis=1)
    p_aug = jnp.concatenate([p, linpad], axis=1)
    bias_pad = jnp.concatenate([bias.astype(jnp.float32), jnp.zeros((D - 1,), jnp.float32)])
    out = _sc_kernel(p_aug, idx, bias_pad)
    return out.reshape(B, 1)
